# baseline (speedup 1.0000x reference)
import functools

import jax
import jax.numpy as jnp
from jax import lax
from jax.experimental import pallas as pl
from jax.experimental.pallas import tpu as pltpu
from jax.experimental.pallas import tpu_sc as plsc

_B, _L, _D = 4096, 200, 64
_NC, _NS = 2, 16
_NW = _NC * _NS
_RPW = _B // _NW


def _sc_embed(idx_flat, emb_table, pos_table):
    mesh = plsc.VectorSubcoreMesh(core_axis_name="c", subcore_axis_name="s")

    @functools.partial(
        pl.kernel,
        mesh=mesh,
        out_type=jax.ShapeDtypeStruct((_B * _L, _D), jnp.float32),
        scratch_types=[
            pltpu.VMEM((_L,), jnp.int32),
            pltpu.VMEM((_L, _D), jnp.float32),
            pltpu.VMEM((_L, _D), jnp.float32),
            pltpu.SemaphoreType.DMA,
        ],
        compiler_params=pltpu.CompilerParams(use_tc_tiling_on_sc=False),
    )
    def k(idx_hbm, table_hbm, pos_hbm, out_hbm, idx_v, rows_v, pos_v, gsem):
        wid = lax.axis_index("s") * _NC + lax.axis_index("c")
        base = wid * _RPW
        pltpu.sync_copy(pos_hbm.at[pl.ds(0, _L)], pos_v)

        def row_body(i, carry):
            tok0 = (base + i) * _L
            pltpu.sync_copy(idx_hbm.at[pl.ds(tok0, _L)], idx_v)
            cp0 = pltpu.async_copy(
                table_hbm.at[idx_v.at[pl.ds(0, 128)]],
                rows_v.at[pl.ds(0, 128)], gsem)
            cp1 = pltpu.async_copy(
                table_hbm.at[idx_v.at[pl.ds(128, _L - 128)]],
                rows_v.at[pl.ds(128, _L - 128)], gsem)
            cp0.wait()
            cp1.wait()

            def add_body(r, acc):
                for c in range(_D // 16):
                    sl = pl.ds(c * 16, 16)
                    plsc.addupdate(rows_v.at[r, sl], pos_v[r, sl])
                return acc

            lax.fori_loop(0, _L, add_body, 0)
            pltpu.sync_copy(rows_v, out_hbm.at[pl.ds(tok0, _L)])
            return carry

        lax.fori_loop(0, _RPW, row_body, 0)

    return k(idx_flat, emb_table, pos_table)


def kernel(input, emb_table, pos_table):
    idx_flat = input.reshape(-1)
    out = _sc_embed(idx_flat, emb_table, pos_table)
    return out.reshape(_B, _L, _D)


# speedup vs baseline: 1.2095x; 1.2095x over previous
import functools

import jax
import jax.numpy as jnp
from jax import lax
from jax.experimental import pallas as pl
from jax.experimental.pallas import tpu as pltpu
from jax.experimental.pallas import tpu_sc as plsc

_B, _L, _D = 4096, 200, 64
_NC, _NS = 2, 16
_NW = _NC * _NS
_RPW = _B // _NW
_NSLOT = 4
_G0 = 128
_G1 = _L - _G0


def _sc_embed(idx_flat, emb_table, pos_table):
    mesh = plsc.VectorSubcoreMesh(core_axis_name="c", subcore_axis_name="s")

    @functools.partial(
        pl.kernel,
        mesh=mesh,
        out_type=jax.ShapeDtypeStruct((_B * _L, _D), jnp.float32),
        scratch_types=[
            pltpu.VMEM((_RPW * _L,), jnp.int32),
            pltpu.VMEM((_NSLOT, _L, _D), jnp.float32),
            pltpu.VMEM((_L, _D), jnp.float32),
        ] + [pltpu.SemaphoreType.DMA] * (2 * _NSLOT),
        compiler_params=pltpu.CompilerParams(use_tc_tiling_on_sc=False),
    )
    def k(idx_hbm, table_hbm, pos_hbm, out_hbm, idx_v, rows_v, pos_v, *sems):
        gsem = sems[:_NSLOT]
        osem = sems[_NSLOT:]
        wid = lax.axis_index("s") * _NC + lax.axis_index("c")
        base_tok = wid * (_RPW * _L)

        pltpu.sync_copy(pos_hbm.at[pl.ds(0, _L)], pos_v)
        pltpu.sync_copy(idx_hbm.at[pl.ds(base_tok, _RPW * _L)], idx_v)

        def issue_gather(i, slot):
            off = i * _L
            pltpu.async_copy(
                table_hbm.at[idx_v.at[pl.ds(off, _G0)]],
                rows_v.at[slot, pl.ds(0, _G0)], gsem[slot])
            pltpu.async_copy(
                table_hbm.at[idx_v.at[pl.ds(off + _G0, _G1)]],
                rows_v.at[slot, pl.ds(_G0, _G1)], gsem[slot])

        def drain(sem, slot):
            pltpu.make_async_copy(
                table_hbm.at[pl.ds(0, _L)], rows_v.at[slot], sem).wait()

        for j in range(2):
            issue_gather(j, j)

        def row_body(i2, carry):
            for b in range(_NSLOT):
                i = i2 * _NSLOT + b
                b2 = (b + 2) % _NSLOT
                @pl.when(jnp.logical_and(i >= 2, i + 2 < _RPW))
                def _():
                    drain(osem[b2], b2)

                @pl.when(i + 2 < _RPW)
                def _():
                    issue_gather(i + 2, b2)

                drain(gsem[b], b)

                def add_body(r2, acc):
                    for k8 in range(8):
                        r = r2 * 8 + k8
                        for c in range(_D // 16):
                            sl = pl.ds(c * 16, 16)
                            plsc.addupdate(rows_v.at[b, r, sl], pos_v[r, sl])
                    return acc

                lax.fori_loop(0, _L // 8, add_body, 0)
                pltpu.async_copy(
                    rows_v.at[b],
                    out_hbm.at[pl.ds(base_tok + i * _L, _L)], osem[b])
            return carry

        lax.fori_loop(0, _RPW // _NSLOT, row_body, 0)
        for b in range(_NSLOT):
            drain(osem[b], b)

    return k(idx_flat, emb_table, pos_table)


def kernel(input, emb_table, pos_table):
    idx_flat = input.reshape(-1)
    out = _sc_embed(idx_flat, emb_table, pos_table)
    return out.reshape(_B, _L, _D)


# speedup vs baseline: 1.2119x; 1.0020x over previous
import functools

import jax
import jax.numpy as jnp
from jax import lax
from jax.experimental import pallas as pl
from jax.experimental.pallas import tpu as pltpu
from jax.experimental.pallas import tpu_sc as plsc

_B, _L, _D = 4096, 200, 64
_NC, _NS = 2, 16
_NW = _NC * _NS
_RPW = _B // _NW
_H0 = 104
_H1 = _L - _H0
_NH = 2 * _RPW
_NSLOT = 8
_LEAD = 4


def _sc_embed(idx_flat, emb_table, pos_table):
    mesh = plsc.VectorSubcoreMesh(core_axis_name="c", subcore_axis_name="s")

    @functools.partial(
        pl.kernel,
        mesh=mesh,
        out_type=jax.ShapeDtypeStruct((_B * _L, _D), jnp.float32),
        scratch_types=[
            pltpu.VMEM((_RPW * _L,), jnp.int32),
            pltpu.VMEM((_NSLOT, _H0, _D), jnp.float32),
            pltpu.VMEM((_L, _D), jnp.float32),
        ] + [pltpu.SemaphoreType.DMA] * (2 * _NSLOT),
        compiler_params=pltpu.CompilerParams(use_tc_tiling_on_sc=False),
    )
    def k(idx_hbm, table_hbm, pos_hbm, out_hbm, idx_v, rows_v, pos_v, *sems):
        gsem = sems[:_NSLOT]
        osem = sems[_NSLOT:]
        wid = lax.axis_index("s") * _NC + lax.axis_index("c")
        base_tok = wid * (_RPW * _L)

        pltpu.sync_copy(pos_hbm.at[pl.ds(0, _L)], pos_v)
        pltpu.sync_copy(idx_hbm.at[pl.ds(base_tok, _RPW * _L)], idx_v)

        def issue_gather(j, par, slot):
            n = _H0 if par == 0 else _H1
            off = (j // 2) * _L + par * _H0
            pltpu.async_copy(
                table_hbm.at[idx_v.at[pl.ds(off, n)]],
                rows_v.at[slot, pl.ds(0, n)], gsem[slot])

        def drain(sem, slot, n):
            pltpu.make_async_copy(
                table_hbm.at[pl.ds(0, n)],
                rows_v.at[slot, pl.ds(0, n)], sem).wait()

        for j in range(_LEAD):
            issue_gather(j, j % 2, j)

        def body(i2, carry):
            for h in range(_NSLOT):
                j = i2 * _NSLOT + h
                par = h % 2
                n = _H0 if par == 0 else _H1
                s2 = (h + _LEAD) % _NSLOT
                par2 = (h + _LEAD) % 2
                n2 = _H0 if par2 == 0 else _H1

                @pl.when(jnp.logical_and(j >= _LEAD, j + _LEAD < _NH))
                def _():
                    drain(osem[s2], s2, n2)

                @pl.when(j + _LEAD < _NH)
                def _():
                    issue_gather(j + _LEAD, par2, s2)

                drain(gsem[h], h, n)

                pos_base = par * _H0

                def add_body(r2, acc):
                    for k8 in range(8):
                        r = r2 * 8 + k8
                        for c in range(_D // 16):
                            sl = pl.ds(c * 16, 16)
                            plsc.addupdate(rows_v.at[h, r, sl],
                                           pos_v[pos_base + r, sl])
                    return acc

                lax.fori_loop(0, n // 8, add_body, 0)

                out_off = base_tok + (j // 2) * _L + par * _H0
                pltpu.async_copy(
                    rows_v.at[h, pl.ds(0, n)],
                    out_hbm.at[pl.ds(out_off, n)], osem[h])
            return carry

        lax.fori_loop(0, _NH // _NSLOT, body, 0)
        for h in range(_NSLOT):
            drain(osem[h], h, _H0 if (h % 2) == 0 else _H1)

    return k(idx_flat, emb_table, pos_table)


def kernel(input, emb_table, pos_table):
    idx_flat = input.reshape(-1)
    out = _sc_embed(idx_flat, emb_table, pos_table)
    return out.reshape(_B, _L, _D)


# speedup vs baseline: 1.6834x; 1.3890x over previous
import functools

import jax
import jax.numpy as jnp
import numpy as np
from jax import lax
from jax.experimental import pallas as pl
from jax.experimental.pallas import tpu as pltpu
from jax.experimental.pallas import tpu_sc as plsc

_B, _L, _D = 4096, 200, 64
_V = 1000000
_NC, _NS = 2, 16
_FPC = _D // _NC
_NLG, _NBH = 8, 2
_LPT = _L // _NLG
_BPT = _B // _NBH
_EPT = _LPT * _BPT
_LCH = 5
_ECH = _LCH * _BPT
_VMAIN = 999936
_RCH = 62464
_REM = _VMAIN - 16 * _RCH


def _sc_embed_t(table_t, idx_t, posx, tail_t):
    mesh = plsc.VectorSubcoreMesh(core_axis_name="c", subcore_axis_name="s")

    @functools.partial(
        pl.kernel,
        mesh=mesh,
        out_type=jax.ShapeDtypeStruct((_L, _D, _B), jnp.float32),
        scratch_types=[
            pltpu.VMEM((_EPT,), jnp.int32),
            pltpu.VMEM((_ECH,), jnp.float32),
            pltpu.VMEM((32, 128), jnp.float32),
            pltpu.VMEM_SHARED((_VMAIN + 128,), jnp.float32),
            pltpu.SemaphoreType.DMA,
            pltpu.SemaphoreType.DMA,
            pltpu.SemaphoreType.DMA,
        ],
        compiler_params=pltpu.CompilerParams(
            use_tc_tiling_on_sc=True, needs_layout_passes=False),
    )
    def k(table_hbm, idx_hbm, posx_hbm, tail_hbm, out_hbm,
          idx_v, res_v, posx_v, row_sh, gsem, osem, isem):
        c = lax.axis_index("c")
        s = lax.axis_index("s")
        lg = s // _NBH
        bh = s % _NBH
        l0 = lg * _LPT
        b0 = bh * _BPT

        def stage_idx(i, carry):
            pltpu.async_copy(
                idx_hbm.at[l0 + i, pl.ds(b0, _BPT)],
                idx_v.at[pl.ds(i * _BPT, _BPT)], isem)
            return carry

        lax.fori_loop(0, _LPT, stage_idx, 0)
        pltpu.sync_copy(posx_hbm.at[pl.ds(lg * 32, 32)], posx_v)

        def drain_idx(i, carry):
            pltpu.make_async_copy(
                idx_hbm.at[l0, pl.ds(b0, _BPT)],
                idx_v.at[pl.ds(i * _BPT, _BPT)], isem).wait()
            return carry

        lax.fori_loop(0, _LPT, drain_idx, 0)

        def drain_out():
            def d(i, carry):
                pltpu.make_async_copy(
                    res_v.at[pl.ds(i * _BPT, _BPT)],
                    out_hbm.at[l0, 0, pl.ds(b0, _BPT)], osem).wait()
                return carry

            lax.fori_loop(0, _LCH, d, 0)

        def feat_body(fl, carry):
            f = c * _FPC + fl
            plsc.subcore_barrier()
            row_off = s * _RCH
            pltpu.sync_copy(
                table_hbm.at[f, pl.ds(row_off, _RCH)],
                row_sh.at[pl.ds(row_off, _RCH)])

            @pl.when(s == _NS - 1)
            def _():
                pltpu.sync_copy(
                    table_hbm.at[f, pl.ds(16 * _RCH, _REM)],
                    row_sh.at[pl.ds(16 * _RCH, _REM)])
                pltpu.sync_copy(
                    tail_hbm.at[f],
                    row_sh.at[pl.ds(_VMAIN, 128)])

            plsc.subcore_barrier()

            for ch in range(_LPT // _LCH):
                if ch == 0:
                    @pl.when(fl >= 1)
                    def _():
                        drain_out()
                else:
                    drain_out()

                pltpu.async_copy(
                    row_sh.at[idx_v.at[pl.ds(ch * _ECH, _ECH)]],
                    res_v, gsem).wait()

                for il in range(_LCH):
                    li = ch * _LCH + il
                    sp = plsc.load_gather(
                        posx_v, [jnp.full((16,), li, jnp.int32),
                                 jnp.full((16,), f, jnp.int32)])

                    def add_j(j, acc, il=il, sp=sp):
                        for k16 in range(16):
                            plsc.addupdate(
                                res_v.at[pl.ds(
                                    il * _BPT + j * 256 + k16 * 16, 16)],
                                sp)
                        return acc

                    lax.fori_loop(0, _BPT // 256, add_j, 0)

                for il in range(_LCH):
                    li = ch * _LCH + il
                    pltpu.async_copy(
                        res_v.at[pl.ds(il * _BPT, _BPT)],
                        out_hbm.at[l0 + li, f, pl.ds(b0, _BPT)], osem)
            return carry

        lax.fori_loop(0, _FPC, feat_body, 0)
        drain_out()

    return k(table_t, idx_t, posx, tail_t)


_SLOT_L = np.minimum((np.arange(256) // 32) * _LPT
                     + np.minimum(np.arange(256) % 32, _LPT - 1), _L - 1)


def kernel(input, emb_table, pos_table):
    tail_t = jnp.pad(emb_table[_VMAIN:], ((0, 128 - (_V - _VMAIN)), (0, 0))).T
    posx = jnp.pad(pos_table[:_L], ((0, 0), (0, 128 - _D)))[_SLOT_L]
    out_t = _sc_embed_t(emb_table.T, input.T, posx, tail_t)
    return out_t.transpose(2, 0, 1)


# speedup vs baseline: 1.9466x; 1.1563x over previous
import functools

import jax
import jax.numpy as jnp
import numpy as np
from jax import lax
from jax.experimental import pallas as pl
from jax.experimental.pallas import tpu as pltpu
from jax.experimental.pallas import tpu_sc as plsc

_B, _L, _D = 4096, 200, 64
_V = 1000000
_NC, _NS = 2, 16
_FPC = _D // _NC
_NLG, _NBH = 8, 2
_LPT = _L // _NLG
_BPT = _B // _NBH
_EPT = _LPT * _BPT
_LCH = 5
_ECH = _LCH * _BPT
_VMAIN = 999936
_RCH = 62464
_REM = _VMAIN - 16 * _RCH


def _sc_embed_t(table_t, idx_t, posx, tail_t):
    mesh = plsc.VectorSubcoreMesh(core_axis_name="c", subcore_axis_name="s")

    @functools.partial(
        pl.kernel,
        mesh=mesh,
        out_type=jax.ShapeDtypeStruct((_L, _D, _B), jnp.float32),
        scratch_types=[
            pltpu.VMEM((_EPT,), jnp.int32),
            pltpu.VMEM((_ECH,), jnp.float32),
            pltpu.VMEM((32, 128), jnp.float32),
            pltpu.VMEM_SHARED((_VMAIN + 128,), jnp.float32),
            pltpu.SemaphoreType.DMA,
            pltpu.SemaphoreType.DMA,
            pltpu.SemaphoreType.DMA,
        ],
        compiler_params=pltpu.CompilerParams(
            use_tc_tiling_on_sc=True, needs_layout_passes=False),
    )
    def k(table_hbm, idx_hbm, posx_hbm, tail_hbm, out_hbm,
          idx_v, res_v, posx_v, row_sh, gsem, osem, isem):
        c = lax.axis_index("c")
        s = lax.axis_index("s")
        lg = s // _NBH
        bh = s % _NBH
        l0 = lg * _LPT
        b0 = bh * _BPT

        def stage_idx(i, carry):
            pltpu.async_copy(
                idx_hbm.at[l0 + i, pl.ds(b0, _BPT)],
                idx_v.at[pl.ds(i * _BPT, _BPT)], isem)
            return carry

        lax.fori_loop(0, _LPT, stage_idx, 0)
        pltpu.sync_copy(posx_hbm.at[pl.ds(lg * 32, 32)], posx_v)

        def drain_idx(i, carry):
            pltpu.make_async_copy(
                idx_hbm.at[l0, pl.ds(b0, _BPT)],
                idx_v.at[pl.ds(i * _BPT, _BPT)], isem).wait()
            return carry

        lax.fori_loop(0, _LPT, drain_idx, 0)

        def drain_out_one(il):
            pltpu.make_async_copy(
                res_v.at[pl.ds(il * _BPT, _BPT)],
                out_hbm.at[l0, 0, pl.ds(b0, _BPT)], osem).wait()

        def feat_body(fl, carry):
            f = c * _FPC + fl
            plsc.subcore_barrier()
            row_off = s * _RCH
            pltpu.sync_copy(
                table_hbm.at[f, pl.ds(row_off, _RCH)],
                row_sh.at[pl.ds(row_off, _RCH)])

            @pl.when(s == _NS - 1)
            def _():
                pltpu.sync_copy(
                    table_hbm.at[f, pl.ds(16 * _RCH, _REM)],
                    row_sh.at[pl.ds(16 * _RCH, _REM)])
                pltpu.sync_copy(
                    tail_hbm.at[f],
                    row_sh.at[pl.ds(_VMAIN, 128)])

            plsc.subcore_barrier()

            for ch in range(_LPT // _LCH):
                for il in range(_LCH):
                    li = ch * _LCH + il
                    if ch == 0:
                        @pl.when(fl >= 1)
                        def _(il=il):
                            drain_out_one(il)
                    else:
                        drain_out_one(il)
                    pltpu.async_copy(
                        row_sh.at[idx_v.at[pl.ds(li * _BPT, _BPT)]],
                        res_v.at[pl.ds(il * _BPT, _BPT)], gsem)

                for il in range(_LCH):
                    li = ch * _LCH + il
                    pltpu.make_async_copy(
                        row_sh.at[idx_v.at[pl.ds(li * _BPT, _BPT)]],
                        res_v.at[pl.ds(il * _BPT, _BPT)], gsem).wait()
                    sp = plsc.load_gather(
                        posx_v, [jnp.full((16,), li, jnp.int32),
                                 jnp.full((16,), f, jnp.int32)])

                    def add_j(j, acc, il=il, sp=sp):
                        for k16 in range(16):
                            plsc.addupdate(
                                res_v.at[pl.ds(
                                    il * _BPT + j * 256 + k16 * 16, 16)],
                                sp)
                        return acc

                    lax.fori_loop(0, _BPT // 256, add_j, 0)
                    pltpu.async_copy(
                        res_v.at[pl.ds(il * _BPT, _BPT)],
                        out_hbm.at[l0 + li, f, pl.ds(b0, _BPT)], osem)
            return carry

        lax.fori_loop(0, _FPC, feat_body, 0)
        for il in range(_LCH):
            drain_out_one(il)

    return k(table_t, idx_t, posx, tail_t)


_SLOT_L = np.minimum((np.arange(256) // 32) * _LPT
                     + np.minimum(np.arange(256) % 32, _LPT - 1), _L - 1)


def kernel(input, emb_table, pos_table):
    tail_t = jnp.pad(emb_table[_VMAIN:], ((0, 128 - (_V - _VMAIN)), (0, 0))).T
    posx = jnp.pad(pos_table[:_L], ((0, 0), (0, 128 - _D)))[_SLOT_L]
    out_t = _sc_embed_t(emb_table.T, input.T, posx, tail_t)
    return out_t.transpose(2, 0, 1)
